# baseline (speedup 1.0000x reference)
import functools

import jax
import jax.numpy as jnp
from jax.experimental import pallas as pl
from jax.experimental.pallas import tpu as pltpu

_NCLS = 19
_PAD = 32
_TEMP = 0.1


def _body(
    fs_ref,
    ft_ref,
    lab_ref,
    out_ref,
    bufs,
    buft,
    sems,
    lab_scr,
    acc_s,
    acc_t,
    cnt,
    *,
    nsteps,
    h_per_b,
    ratio,
):
    i = pl.program_id(0)
    h = i % h_per_b
    b = i // h_per_b
    slot = i % 2
    f32 = jnp.float32
    bf16 = jnp.bfloat16

    @pl.when(i == 0)
    def _init():
        acc_s[...] = jnp.zeros_like(acc_s)
        acc_t[...] = jnp.zeros_like(acc_t)
        cnt[...] = jnp.zeros_like(cnt)
        pltpu.make_async_copy(fs_ref.at[0, :, 0, :], bufs.at[0], sems.at[0, 0]).start()
        pltpu.make_async_copy(ft_ref.at[0, :, 0, :], buft.at[0], sems.at[0, 1]).start()

    @pl.when(i + 1 < nsteps)
    def _prefetch():
        nslot = (i + 1) % 2
        nh = (i + 1) % h_per_b
        nb = (i + 1) // h_per_b
        pltpu.make_async_copy(
            fs_ref.at[nb, :, nh, :], bufs.at[nslot], sems.at[nslot, 0]
        ).start()
        pltpu.make_async_copy(
            ft_ref.at[nb, :, nh, :], buft.at[nslot], sems.at[nslot, 1]
        ).start()

    @pl.when(h == 0)
    def _prep_labels():
        lab = lab_ref[...].astype(bf16)
        lh, lw = lab.shape
        w = lw // ratio
        hh = lh // ratio
        jc = jax.lax.broadcasted_iota(jnp.int32, (lw, w), 0)
        wc = jax.lax.broadcasted_iota(jnp.int32, (lw, w), 1)
        sel_col = (jc == ratio * wc).astype(bf16)
        b2 = jax.lax.dot(lab, sel_col, preferred_element_type=f32).astype(
            bf16
        )
        rr = jax.lax.broadcasted_iota(jnp.int32, (hh, lh), 0)
        jr = jax.lax.broadcasted_iota(jnp.int32, (hh, lh), 1)
        sel_row = (jr == ratio * rr).astype(bf16)
        lab_scr[...] = jax.lax.dot(sel_row, b2, preferred_element_type=f32)

    pltpu.make_async_copy(fs_ref.at[b, :, h, :], bufs.at[slot], sems.at[slot, 0]).wait()
    pltpu.make_async_copy(ft_ref.at[b, :, h, :], buft.at[slot], sems.at[slot, 1]).wait()

    x_s = bufs[slot]
    x_t = buft[slot]
    w = x_s.shape[1]

    row = lab_scr[pl.ds(h, 1), :]
    kio = jax.lax.broadcasted_iota(jnp.int32, (_PAD, w), 0).astype(f32)
    oh_t_f = (jnp.broadcast_to(row, (_PAD, w)) == kio).astype(f32)
    oh_t = oh_t_f.astype(bf16)

    s_hi = x_s.astype(bf16)
    s_lo = (x_s - s_hi.astype(f32)).astype(bf16)
    t_hi = x_t.astype(bf16)
    t_lo = (x_t - t_hi.astype(f32)).astype(bf16)

    dims = (((1,), (1,)), ((), ()))
    acc_s[...] += jax.lax.dot_general(
        s_hi, oh_t, dims, preferred_element_type=f32
    ) + jax.lax.dot_general(s_lo, oh_t, dims, preferred_element_type=f32)
    acc_t[...] += jax.lax.dot_general(
        t_hi, oh_t, dims, preferred_element_type=f32
    ) + jax.lax.dot_general(t_lo, oh_t, dims, preferred_element_type=f32)
    cnt[...] += oh_t_f

    @pl.when(i == nsteps - 1)
    def _fin():
        hp = jax.lax.Precision.HIGHEST
        counts = jnp.sum(cnt[...], axis=1)
        present = counts > 0.0
        denom = jnp.where(present, counts, 1.0)
        mean_s = acc_s[...] / denom[None, :]
        mean_t = acc_t[...] / denom[None, :]
        ns = jnp.sqrt(jnp.sum(mean_s * mean_s, axis=0, keepdims=True))
        nt = jnp.sqrt(jnp.sum(mean_t * mean_t, axis=0, keepdims=True))
        s_n = jnp.where(present[None, :], mean_s / jnp.maximum(ns, 1e-12), 0.0)
        t_n = jnp.where(present[None, :], mean_t / jnp.maximum(nt, 1e-12), 0.0)
        logits = (
            jax.lax.dot_general(s_n, t_n, (((0,), (0,)), ((), ())), precision=hp)
            / _TEMP
        )
        ri = jax.lax.broadcasted_iota(jnp.int32, (_PAD, _PAD), 0)
        ci = jax.lax.broadcasted_iota(jnp.int32, (_PAD, _PAD), 1)
        e = jnp.where(ci < _NCLS, jnp.exp(logits), 0.0)
        row_sum = jnp.sum(e, axis=1)
        diag = jnp.sum(jnp.where(ri == ci, logits, 0.0), axis=1)
        per_cls = jnp.log(row_sum) - diag
        n_present = jnp.maximum(jnp.sum(jnp.where(present, 1.0, 0.0)), 1.0)
        loss = jnp.sum(jnp.where(present, per_cls, 0.0)) / n_present
        out_ref[...] = jnp.broadcast_to(loss, (1, 1))


@jax.jit
def kernel(features_s, features_t, labels):
    B, C, H, W = features_s.shape
    Lh, Lw = labels.shape[1], labels.shape[2]
    ratio = Lh // H
    nsteps = B * H

    out = pl.pallas_call(
        functools.partial(_body, nsteps=nsteps, h_per_b=H, ratio=ratio),
        grid=(nsteps,),
        in_specs=[
            pl.BlockSpec(memory_space=pltpu.MemorySpace.HBM),
            pl.BlockSpec(memory_space=pltpu.MemorySpace.HBM),
            pl.BlockSpec((None, Lh, Lw), lambda i: (i // H, 0, 0)),
        ],
        out_specs=pl.BlockSpec((1, 1), lambda i: (0, 0)),
        out_shape=jax.ShapeDtypeStruct((1, 1), jnp.float32),
        scratch_shapes=[
            pltpu.VMEM((2, C, W), jnp.float32),
            pltpu.VMEM((2, C, W), jnp.float32),
            pltpu.SemaphoreType.DMA((2, 2)),
            pltpu.VMEM((H, W), jnp.float32),
            pltpu.VMEM((C, _PAD), jnp.float32),
            pltpu.VMEM((C, _PAD), jnp.float32),
            pltpu.VMEM((_PAD, W), jnp.float32),
        ],
    )(features_s, features_t, labels)
    return out[0, 0]


# speedup vs baseline: 2.1582x; 2.1582x over previous
import functools

import jax
import jax.numpy as jnp
from jax.experimental import pallas as pl
from jax.experimental.pallas import tpu as pltpu

_NCLS = 19
_PAD = 32
_TEMP = 0.1


def _body(
    fs_ref,
    ft_ref,
    lab_ref,
    out_ref,
    bufs,
    buft,
    sems,
    lab_scr,
    acc_s,
    acc_t,
    cnt,
    *,
    nsteps,
    h_per_b,
    ratio,
):
    i = pl.program_id(0)
    h = i % h_per_b
    b = i // h_per_b
    nbuf = bufs.shape[0]
    slot = i % nbuf
    f32 = jnp.float32
    bf16 = jnp.bfloat16

    @pl.when(i == 0)
    def _init():
        acc_s[...] = jnp.zeros_like(acc_s)
        acc_t[...] = jnp.zeros_like(acc_t)
        cnt[...] = jnp.zeros_like(cnt)
        for j in range(nbuf - 1):
            jh = j % h_per_b
            jb = j // h_per_b
            pltpu.make_async_copy(
                fs_ref.at[jb, :, jh, :], bufs.at[j], sems.at[j, 0]
            ).start()
            pltpu.make_async_copy(
                ft_ref.at[jb, :, jh, :], buft.at[j], sems.at[j, 1]
            ).start()

    nxt = i + nbuf - 1

    @pl.when(nxt < nsteps)
    def _prefetch():
        nslot = nxt % nbuf
        nh = nxt % h_per_b
        nb = nxt // h_per_b
        pltpu.make_async_copy(
            fs_ref.at[nb, :, nh, :], bufs.at[nslot], sems.at[nslot, 0]
        ).start()
        pltpu.make_async_copy(
            ft_ref.at[nb, :, nh, :], buft.at[nslot], sems.at[nslot, 1]
        ).start()

    @pl.when(h == 0)
    def _prep_labels():
        lab = lab_ref[...].astype(bf16)
        lh, lw = lab.shape
        w = lw // ratio
        hh = lh // ratio
        jc = jax.lax.broadcasted_iota(jnp.int32, (lw, w), 0)
        wc = jax.lax.broadcasted_iota(jnp.int32, (lw, w), 1)
        sel_col = (jc == ratio * wc).astype(bf16)
        b2 = jax.lax.dot(lab, sel_col, preferred_element_type=f32).astype(
            bf16
        )
        rr = jax.lax.broadcasted_iota(jnp.int32, (hh, lh), 0)
        jr = jax.lax.broadcasted_iota(jnp.int32, (hh, lh), 1)
        sel_row = (jr == ratio * rr).astype(bf16)
        lab_scr[...] = jax.lax.dot(sel_row, b2, preferred_element_type=f32)

    pltpu.make_async_copy(fs_ref.at[b, :, h, :], bufs.at[slot], sems.at[slot, 0]).wait()
    pltpu.make_async_copy(ft_ref.at[b, :, h, :], buft.at[slot], sems.at[slot, 1]).wait()

    x_s = bufs[slot]
    x_t = buft[slot]
    w = x_s.shape[1]

    row = lab_scr[pl.ds(h, 1), :]
    kio = jax.lax.broadcasted_iota(jnp.int32, (_PAD, w), 0).astype(f32)
    oh_t_f = (jnp.broadcast_to(row, (_PAD, w)) == kio).astype(f32)
    oh_t = oh_t_f.astype(bf16)

    s_hi = x_s.astype(bf16)
    s_lo = (x_s - s_hi.astype(f32)).astype(bf16)
    t_hi = x_t.astype(bf16)
    t_lo = (x_t - t_hi.astype(f32)).astype(bf16)

    dims = (((1,), (1,)), ((), ()))
    acc_s[...] += jax.lax.dot_general(
        s_hi, oh_t, dims, preferred_element_type=f32
    ) + jax.lax.dot_general(s_lo, oh_t, dims, preferred_element_type=f32)
    acc_t[...] += jax.lax.dot_general(
        t_hi, oh_t, dims, preferred_element_type=f32
    ) + jax.lax.dot_general(t_lo, oh_t, dims, preferred_element_type=f32)
    cnt[...] += oh_t_f

    @pl.when(i == nsteps - 1)
    def _fin():
        hp = jax.lax.Precision.HIGHEST
        counts = jnp.sum(cnt[...], axis=1)
        present = counts > 0.0
        denom = jnp.where(present, counts, 1.0)
        mean_s = acc_s[...] / denom[None, :]
        mean_t = acc_t[...] / denom[None, :]
        ns = jnp.sqrt(jnp.sum(mean_s * mean_s, axis=0, keepdims=True))
        nt = jnp.sqrt(jnp.sum(mean_t * mean_t, axis=0, keepdims=True))
        s_n = jnp.where(present[None, :], mean_s / jnp.maximum(ns, 1e-12), 0.0)
        t_n = jnp.where(present[None, :], mean_t / jnp.maximum(nt, 1e-12), 0.0)
        logits = (
            jax.lax.dot_general(s_n, t_n, (((0,), (0,)), ((), ())), precision=hp)
            / _TEMP
        )
        ri = jax.lax.broadcasted_iota(jnp.int32, (_PAD, _PAD), 0)
        ci = jax.lax.broadcasted_iota(jnp.int32, (_PAD, _PAD), 1)
        e = jnp.where(ci < _NCLS, jnp.exp(logits), 0.0)
        row_sum = jnp.sum(e, axis=1)
        diag = jnp.sum(jnp.where(ri == ci, logits, 0.0), axis=1)
        per_cls = jnp.log(row_sum) - diag
        n_present = jnp.maximum(jnp.sum(jnp.where(present, 1.0, 0.0)), 1.0)
        loss = jnp.sum(jnp.where(present, per_cls, 0.0)) / n_present
        out_ref[...] = jnp.broadcast_to(loss, (1, 1))


@jax.jit
def kernel(features_s, features_t, labels):
    B, C, H, W = features_s.shape
    Lh, Lw = labels.shape[1], labels.shape[2]
    ratio = Lh // H
    nsteps = B * H

    out = pl.pallas_call(
        functools.partial(_body, nsteps=nsteps, h_per_b=H, ratio=ratio),
        grid=(nsteps,),
        in_specs=[
            pl.BlockSpec(memory_space=pltpu.MemorySpace.HBM),
            pl.BlockSpec(memory_space=pltpu.MemorySpace.HBM),
            pl.BlockSpec((None, Lh, Lw), lambda i: (i // H, 0, 0)),
        ],
        out_specs=pl.BlockSpec((1, 1), lambda i: (0, 0)),
        out_shape=jax.ShapeDtypeStruct((1, 1), jnp.float32),
        scratch_shapes=[
            pltpu.VMEM((8, C, W), jnp.float32),
            pltpu.VMEM((8, C, W), jnp.float32),
            pltpu.SemaphoreType.DMA((8, 2)),
            pltpu.VMEM((H, W), jnp.float32),
            pltpu.VMEM((C, _PAD), jnp.float32),
            pltpu.VMEM((C, _PAD), jnp.float32),
            pltpu.VMEM((_PAD, W), jnp.float32),
        ],
    )(features_s, features_t, labels)
    return out[0, 0]


# speedup vs baseline: 2.1599x; 1.0008x over previous
import functools

import jax
import jax.numpy as jnp
from jax.experimental import pallas as pl
from jax.experimental.pallas import tpu as pltpu

_NCLS = 19
_PAD = 32
_TEMP = 0.1


def _body(
    fs_ref,
    ft_ref,
    lab_ref,
    out_ref,
    bufs,
    buft,
    sems,
    lab_scr,
    acc_s,
    acc_t,
    cnt,
    *,
    nsteps,
    h_per_b,
    ratio,
):
    i = pl.program_id(0)
    h = i % h_per_b
    b = i // h_per_b
    nbuf = bufs.shape[0]
    slot = i % nbuf
    f32 = jnp.float32
    bf16 = jnp.bfloat16

    @pl.when(i == 0)
    def _init():
        acc_s[...] = jnp.zeros_like(acc_s)
        acc_t[...] = jnp.zeros_like(acc_t)
        cnt[...] = jnp.zeros_like(cnt)
        for j in range(nbuf - 1):
            jh = j % h_per_b
            jb = j // h_per_b
            pltpu.make_async_copy(
                fs_ref.at[jb, :, jh, :], bufs.at[j], sems.at[j, 0]
            ).start()
            pltpu.make_async_copy(
                ft_ref.at[jb, :, jh, :], buft.at[j], sems.at[j, 1]
            ).start()

    nxt = i + nbuf - 1

    @pl.when(nxt < nsteps)
    def _prefetch():
        nslot = nxt % nbuf
        nh = nxt % h_per_b
        nb = nxt // h_per_b
        pltpu.make_async_copy(
            fs_ref.at[nb, :, nh, :], bufs.at[nslot], sems.at[nslot, 0]
        ).start()
        pltpu.make_async_copy(
            ft_ref.at[nb, :, nh, :], buft.at[nslot], sems.at[nslot, 1]
        ).start()

    @pl.when(h == 0)
    def _prep_labels():
        lab = lab_ref[...].astype(bf16)
        lh, lw = lab.shape
        w = lw // ratio
        hh = lh // ratio
        jc = jax.lax.broadcasted_iota(jnp.int32, (lw, w), 0)
        wc = jax.lax.broadcasted_iota(jnp.int32, (lw, w), 1)
        sel_col = (jc == ratio * wc).astype(bf16)
        b2 = jax.lax.dot(lab, sel_col, preferred_element_type=f32).astype(
            bf16
        )
        rr = jax.lax.broadcasted_iota(jnp.int32, (hh, lh), 0)
        jr = jax.lax.broadcasted_iota(jnp.int32, (hh, lh), 1)
        sel_row = (jr == ratio * rr).astype(bf16)
        lab_scr[...] = jax.lax.dot(sel_row, b2, preferred_element_type=f32)

    pltpu.make_async_copy(fs_ref.at[b, :, h, :], bufs.at[slot], sems.at[slot, 0]).wait()
    pltpu.make_async_copy(ft_ref.at[b, :, h, :], buft.at[slot], sems.at[slot, 1]).wait()

    x_s = bufs[slot]
    x_t = buft[slot]
    w = x_s.shape[1]

    row = lab_scr[pl.ds(h, 1), :]
    kio = jax.lax.broadcasted_iota(jnp.int32, (_PAD, w), 0).astype(f32)
    oh_t_f = (jnp.broadcast_to(row, (_PAD, w)) == kio).astype(f32)
    oh_t = oh_t_f.astype(bf16)

    s_hi = x_s.astype(bf16)
    s_lo = (x_s - s_hi.astype(f32)).astype(bf16)
    t_hi = x_t.astype(bf16)
    t_lo = (x_t - t_hi.astype(f32)).astype(bf16)

    dims = (((1,), (1,)), ((), ()))
    acc_s[...] += jax.lax.dot_general(
        s_hi, oh_t, dims, preferred_element_type=f32
    ) + jax.lax.dot_general(s_lo, oh_t, dims, preferred_element_type=f32)
    acc_t[...] += jax.lax.dot_general(
        t_hi, oh_t, dims, preferred_element_type=f32
    ) + jax.lax.dot_general(t_lo, oh_t, dims, preferred_element_type=f32)
    cnt[...] += oh_t_f

    @pl.when(i == nsteps - 1)
    def _fin():
        hp = jax.lax.Precision.HIGHEST
        counts = jnp.sum(cnt[...], axis=1)
        present = counts > 0.0
        denom = jnp.where(present, counts, 1.0)
        mean_s = acc_s[...] / denom[None, :]
        mean_t = acc_t[...] / denom[None, :]
        ns = jnp.sqrt(jnp.sum(mean_s * mean_s, axis=0, keepdims=True))
        nt = jnp.sqrt(jnp.sum(mean_t * mean_t, axis=0, keepdims=True))
        s_n = jnp.where(present[None, :], mean_s / jnp.maximum(ns, 1e-12), 0.0)
        t_n = jnp.where(present[None, :], mean_t / jnp.maximum(nt, 1e-12), 0.0)
        logits = (
            jax.lax.dot_general(s_n, t_n, (((0,), (0,)), ((), ())), precision=hp)
            / _TEMP
        )
        ri = jax.lax.broadcasted_iota(jnp.int32, (_PAD, _PAD), 0)
        ci = jax.lax.broadcasted_iota(jnp.int32, (_PAD, _PAD), 1)
        e = jnp.where(ci < _NCLS, jnp.exp(logits), 0.0)
        row_sum = jnp.sum(e, axis=1)
        diag = jnp.sum(jnp.where(ri == ci, logits, 0.0), axis=1)
        per_cls = jnp.log(row_sum) - diag
        n_present = jnp.maximum(jnp.sum(jnp.where(present, 1.0, 0.0)), 1.0)
        loss = jnp.sum(jnp.where(present, per_cls, 0.0)) / n_present
        out_ref[...] = jnp.broadcast_to(loss, (1, 1))


@jax.jit
def kernel(features_s, features_t, labels):
    B, C, H, W = features_s.shape
    Lh, Lw = labels.shape[1], labels.shape[2]
    ratio = Lh // H
    nsteps = B * H

    out = pl.pallas_call(
        functools.partial(_body, nsteps=nsteps, h_per_b=H, ratio=ratio),
        grid=(nsteps,),
        in_specs=[
            pl.BlockSpec(memory_space=pltpu.MemorySpace.HBM),
            pl.BlockSpec(memory_space=pltpu.MemorySpace.HBM),
            pl.BlockSpec((None, Lh, Lw), lambda i: (i // H, 0, 0)),
        ],
        out_specs=pl.BlockSpec((1, 1), lambda i: (0, 0)),
        out_shape=jax.ShapeDtypeStruct((1, 1), jnp.float32),
        scratch_shapes=[
            pltpu.VMEM((16, C, W), jnp.float32),
            pltpu.VMEM((16, C, W), jnp.float32),
            pltpu.SemaphoreType.DMA((16, 2)),
            pltpu.VMEM((H, W), jnp.float32),
            pltpu.VMEM((C, _PAD), jnp.float32),
            pltpu.VMEM((C, _PAD), jnp.float32),
            pltpu.VMEM((_PAD, W), jnp.float32),
        ],
    )(features_s, features_t, labels)
    return out[0, 0]


# speedup vs baseline: 4.6991x; 2.1757x over previous
import functools

import jax
import jax.numpy as jnp
from jax.experimental import pallas as pl
from jax.experimental.pallas import tpu as pltpu

_NCLS = 19
_PAD = 32
_TEMP = 0.1


def _body(
    fs_ref,
    ft_ref,
    lab_ref,
    out_ref,
    bufs,
    buft,
    sems,
    lab_scr,
    acc_s,
    acc_t,
    cnt,
    *,
    nsteps,
    h_per_b,
    ratio,
):
    i = pl.program_id(0)
    nbuf, rows = bufs.shape[0], bufs.shape[1]
    steps_per_b = h_per_b // rows
    h0 = (i % steps_per_b) * rows
    b = i // steps_per_b
    slot = i % nbuf
    f32 = jnp.float32
    bf16 = jnp.bfloat16

    def _start_group(step, slot_idx):
        gh = (step % steps_per_b) * rows
        gb = step // steps_per_b
        for r in range(rows):
            pltpu.make_async_copy(
                fs_ref.at[gb, :, gh + r, :], bufs.at[slot_idx, r], sems.at[slot_idx, r, 0]
            ).start()
            pltpu.make_async_copy(
                ft_ref.at[gb, :, gh + r, :], buft.at[slot_idx, r], sems.at[slot_idx, r, 1]
            ).start()

    @pl.when(i == 0)
    def _init():
        acc_s[...] = jnp.zeros_like(acc_s)
        acc_t[...] = jnp.zeros_like(acc_t)
        cnt[...] = jnp.zeros_like(cnt)
        for j in range(nbuf - 1):
            _start_group(j, j)

    nxt = i + nbuf - 1

    @pl.when(nxt < nsteps)
    def _prefetch():
        _start_group(nxt, nxt % nbuf)

    @pl.when(i % steps_per_b == 0)
    def _prep_labels():
        lab = lab_ref[...].astype(bf16)
        lh, lw = lab.shape
        w = lw // ratio
        hh = lh // ratio
        jc = jax.lax.broadcasted_iota(jnp.int32, (lw, w), 0)
        wc = jax.lax.broadcasted_iota(jnp.int32, (lw, w), 1)
        sel_col = (jc == ratio * wc).astype(bf16)
        b2 = jax.lax.dot(lab, sel_col, preferred_element_type=f32).astype(
            bf16
        )
        rr = jax.lax.broadcasted_iota(jnp.int32, (hh, lh), 0)
        jr = jax.lax.broadcasted_iota(jnp.int32, (hh, lh), 1)
        sel_row = (jr == ratio * rr).astype(bf16)
        lab_scr[...] = jax.lax.dot(sel_row, b2, preferred_element_type=f32)

    w = bufs.shape[3]
    kio = jax.lax.broadcasted_iota(jnp.int32, (_PAD, w), 0).astype(f32)
    dims = (((1,), (1,)), ((), ()))

    ps = None
    pt = None
    csum = None
    for r in range(rows):
        pltpu.make_async_copy(
            fs_ref.at[b, :, h0 + r, :], bufs.at[slot, r], sems.at[slot, r, 0]
        ).wait()
        pltpu.make_async_copy(
            ft_ref.at[b, :, h0 + r, :], buft.at[slot, r], sems.at[slot, r, 1]
        ).wait()
        x_s = bufs[slot, r]
        x_t = buft[slot, r]

        row = lab_scr[pl.ds(h0 + r, 1), :]
        oh_t_f = (jnp.broadcast_to(row, (_PAD, w)) == kio).astype(f32)
        oh_t = oh_t_f.astype(bf16)

        s_hi = x_s.astype(bf16)
        s_lo = (x_s - s_hi.astype(f32)).astype(bf16)
        t_hi = x_t.astype(bf16)
        t_lo = (x_t - t_hi.astype(f32)).astype(bf16)

        ds = jax.lax.dot_general(
            s_hi, oh_t, dims, preferred_element_type=f32
        ) + jax.lax.dot_general(s_lo, oh_t, dims, preferred_element_type=f32)
        dt = jax.lax.dot_general(
            t_hi, oh_t, dims, preferred_element_type=f32
        ) + jax.lax.dot_general(t_lo, oh_t, dims, preferred_element_type=f32)
        ps = ds if ps is None else ps + ds
        pt = dt if pt is None else pt + dt
        csum = oh_t_f if csum is None else csum + oh_t_f
    acc_s[...] += ps
    acc_t[...] += pt
    cnt[...] += csum

    @pl.when(i == nsteps - 1)
    def _fin():
        hp = jax.lax.Precision.HIGHEST
        counts = jnp.sum(cnt[...], axis=1)
        present = counts > 0.0
        denom = jnp.where(present, counts, 1.0)
        mean_s = acc_s[...] / denom[None, :]
        mean_t = acc_t[...] / denom[None, :]
        ns = jnp.sqrt(jnp.sum(mean_s * mean_s, axis=0, keepdims=True))
        nt = jnp.sqrt(jnp.sum(mean_t * mean_t, axis=0, keepdims=True))
        s_n = jnp.where(present[None, :], mean_s / jnp.maximum(ns, 1e-12), 0.0)
        t_n = jnp.where(present[None, :], mean_t / jnp.maximum(nt, 1e-12), 0.0)
        logits = (
            jax.lax.dot_general(s_n, t_n, (((0,), (0,)), ((), ())), precision=hp)
            / _TEMP
        )
        ri = jax.lax.broadcasted_iota(jnp.int32, (_PAD, _PAD), 0)
        ci = jax.lax.broadcasted_iota(jnp.int32, (_PAD, _PAD), 1)
        e = jnp.where(ci < _NCLS, jnp.exp(logits), 0.0)
        row_sum = jnp.sum(e, axis=1)
        diag = jnp.sum(jnp.where(ri == ci, logits, 0.0), axis=1)
        per_cls = jnp.log(row_sum) - diag
        n_present = jnp.maximum(jnp.sum(jnp.where(present, 1.0, 0.0)), 1.0)
        loss = jnp.sum(jnp.where(present, per_cls, 0.0)) / n_present
        out_ref[...] = jnp.broadcast_to(loss, (1, 1))


@jax.jit
def kernel(features_s, features_t, labels):
    B, C, H, W = features_s.shape
    Lh, Lw = labels.shape[1], labels.shape[2]
    ratio = Lh // H
    rows = 8
    nbuf = 3
    steps_per_b = H // rows
    nsteps = B * steps_per_b

    out = pl.pallas_call(
        functools.partial(_body, nsteps=nsteps, h_per_b=H, ratio=ratio),
        grid=(nsteps,),
        in_specs=[
            pl.BlockSpec(memory_space=pltpu.MemorySpace.HBM),
            pl.BlockSpec(memory_space=pltpu.MemorySpace.HBM),
            pl.BlockSpec((None, Lh, Lw), lambda i: (i // steps_per_b, 0, 0)),
        ],
        out_specs=pl.BlockSpec((1, 1), lambda i: (0, 0)),
        out_shape=jax.ShapeDtypeStruct((1, 1), jnp.float32),
        scratch_shapes=[
            pltpu.VMEM((nbuf, rows, C, W), jnp.float32),
            pltpu.VMEM((nbuf, rows, C, W), jnp.float32),
            pltpu.SemaphoreType.DMA((nbuf, rows, 2)),
            pltpu.VMEM((H, W), jnp.float32),
            pltpu.VMEM((C, _PAD), jnp.float32),
            pltpu.VMEM((C, _PAD), jnp.float32),
            pltpu.VMEM((_PAD, W), jnp.float32),
        ],
    )(features_s, features_t, labels)
    return out[0, 0]


# speedup vs baseline: 5.3638x; 1.1415x over previous
import functools

import jax
import jax.numpy as jnp
from jax.experimental import pallas as pl
from jax.experimental.pallas import tpu as pltpu

_NCLS = 19
_PAD = 32
_TEMP = 0.1


def _body(
    fs_ref,
    ft_ref,
    lab_ref,
    out_ref,
    bufs,
    buft,
    sems,
    lab_scr,
    acc_s,
    acc_t,
    cnt,
    *,
    nsteps,
    h_per_b,
    ratio,
):
    i = pl.program_id(0)
    nbuf, rows = bufs.shape[0], bufs.shape[1]
    steps_per_b = h_per_b // rows
    h0 = (i % steps_per_b) * rows
    b = i // steps_per_b
    slot = i % nbuf
    f32 = jnp.float32
    bf16 = jnp.bfloat16

    def _start_group(step, slot_idx):
        gh = (step % steps_per_b) * rows
        gb = step // steps_per_b
        for r in range(rows):
            pltpu.make_async_copy(
                fs_ref.at[gb, :, gh + r, :], bufs.at[slot_idx, r], sems.at[slot_idx, r, 0]
            ).start()
            pltpu.make_async_copy(
                ft_ref.at[gb, :, gh + r, :], buft.at[slot_idx, r], sems.at[slot_idx, r, 1]
            ).start()

    @pl.when(i == 0)
    def _init():
        acc_s[...] = jnp.zeros_like(acc_s)
        acc_t[...] = jnp.zeros_like(acc_t)
        cnt[...] = jnp.zeros_like(cnt)
        for j in range(nbuf - 1):
            _start_group(j, j)

    nxt = i + nbuf - 1

    @pl.when(nxt < nsteps)
    def _prefetch():
        _start_group(nxt, nxt % nbuf)

    @pl.when(i % steps_per_b == 0)
    def _prep_labels():
        lab = lab_ref[...].astype(bf16)
        lh, lw = lab.shape
        w = lw // ratio
        hh = lh // ratio
        jc = jax.lax.broadcasted_iota(jnp.int32, (lw, w), 0)
        wc = jax.lax.broadcasted_iota(jnp.int32, (lw, w), 1)
        sel_col = (jc == ratio * wc).astype(bf16)
        b2 = jax.lax.dot(lab, sel_col, preferred_element_type=f32).astype(
            bf16
        )
        rr = jax.lax.broadcasted_iota(jnp.int32, (hh, lh), 0)
        jr = jax.lax.broadcasted_iota(jnp.int32, (hh, lh), 1)
        sel_row = (jr == ratio * rr).astype(bf16)
        lab_scr[...] = jax.lax.dot(sel_row, b2, preferred_element_type=f32)

    w = bufs.shape[3]
    kio = jax.lax.broadcasted_iota(jnp.int32, (_PAD, w), 0).astype(f32)
    dims = (((1,), (1,)), ((), ()))

    ps = None
    pt = None
    csum = None
    for r in range(rows):
        pltpu.make_async_copy(
            fs_ref.at[b, :, h0 + r, :], bufs.at[slot, r], sems.at[slot, r, 0]
        ).wait()
        pltpu.make_async_copy(
            ft_ref.at[b, :, h0 + r, :], buft.at[slot, r], sems.at[slot, r, 1]
        ).wait()
        x_s = bufs[slot, r]
        x_t = buft[slot, r]

        row = lab_scr[pl.ds(h0 + r, 1), :]
        oh_t_f = (jnp.broadcast_to(row, (_PAD, w)) == kio).astype(f32)
        oh_t = oh_t_f.astype(bf16)

        s_hi = x_s.astype(bf16)
        s_lo = (x_s - s_hi.astype(f32)).astype(bf16)
        t_hi = x_t.astype(bf16)
        t_lo = (x_t - t_hi.astype(f32)).astype(bf16)

        ds = jax.lax.dot_general(
            s_hi, oh_t, dims, preferred_element_type=f32
        ) + jax.lax.dot_general(s_lo, oh_t, dims, preferred_element_type=f32)
        dt = jax.lax.dot_general(
            t_hi, oh_t, dims, preferred_element_type=f32
        ) + jax.lax.dot_general(t_lo, oh_t, dims, preferred_element_type=f32)
        ps = ds if ps is None else ps + ds
        pt = dt if pt is None else pt + dt
        csum = oh_t_f if csum is None else csum + oh_t_f
    acc_s[...] += ps
    acc_t[...] += pt
    cnt[...] += csum

    @pl.when(i == nsteps - 1)
    def _fin():
        hp = jax.lax.Precision.HIGHEST
        counts = jnp.sum(cnt[...], axis=1)
        present = counts > 0.0
        denom = jnp.where(present, counts, 1.0)
        mean_s = acc_s[...] / denom[None, :]
        mean_t = acc_t[...] / denom[None, :]
        ns = jnp.sqrt(jnp.sum(mean_s * mean_s, axis=0, keepdims=True))
        nt = jnp.sqrt(jnp.sum(mean_t * mean_t, axis=0, keepdims=True))
        s_n = jnp.where(present[None, :], mean_s / jnp.maximum(ns, 1e-12), 0.0)
        t_n = jnp.where(present[None, :], mean_t / jnp.maximum(nt, 1e-12), 0.0)
        logits = (
            jax.lax.dot_general(s_n, t_n, (((0,), (0,)), ((), ())), precision=hp)
            / _TEMP
        )
        ri = jax.lax.broadcasted_iota(jnp.int32, (_PAD, _PAD), 0)
        ci = jax.lax.broadcasted_iota(jnp.int32, (_PAD, _PAD), 1)
        e = jnp.where(ci < _NCLS, jnp.exp(logits), 0.0)
        row_sum = jnp.sum(e, axis=1)
        diag = jnp.sum(jnp.where(ri == ci, logits, 0.0), axis=1)
        per_cls = jnp.log(row_sum) - diag
        n_present = jnp.maximum(jnp.sum(jnp.where(present, 1.0, 0.0)), 1.0)
        loss = jnp.sum(jnp.where(present, per_cls, 0.0)) / n_present
        out_ref[...] = jnp.broadcast_to(loss, (1, 1))


@jax.jit
def kernel(features_s, features_t, labels):
    B, C, H, W = features_s.shape
    Lh, Lw = labels.shape[1], labels.shape[2]
    ratio = Lh // H
    rows = 16
    nbuf = 3
    steps_per_b = H // rows
    nsteps = B * steps_per_b

    out = pl.pallas_call(
        functools.partial(_body, nsteps=nsteps, h_per_b=H, ratio=ratio),
        grid=(nsteps,),
        in_specs=[
            pl.BlockSpec(memory_space=pltpu.MemorySpace.HBM),
            pl.BlockSpec(memory_space=pltpu.MemorySpace.HBM),
            pl.BlockSpec((None, Lh, Lw), lambda i: (i // steps_per_b, 0, 0)),
        ],
        out_specs=pl.BlockSpec((1, 1), lambda i: (0, 0)),
        out_shape=jax.ShapeDtypeStruct((1, 1), jnp.float32),
        scratch_shapes=[
            pltpu.VMEM((nbuf, rows, C, W), jnp.float32),
            pltpu.VMEM((nbuf, rows, C, W), jnp.float32),
            pltpu.SemaphoreType.DMA((nbuf, rows, 2)),
            pltpu.VMEM((H, W), jnp.float32),
            pltpu.VMEM((C, _PAD), jnp.float32),
            pltpu.VMEM((C, _PAD), jnp.float32),
            pltpu.VMEM((_PAD, W), jnp.float32),
        ],
    )(features_s, features_t, labels)
    return out[0, 0]


# speedup vs baseline: 5.6249x; 1.0487x over previous
import functools

import jax
import jax.numpy as jnp
from jax.experimental import pallas as pl
from jax.experimental.pallas import tpu as pltpu

_NCLS = 19
_PAD = 32
_TEMP = 0.1


def _body(
    fs_ref,
    ft_ref,
    lab_ref,
    out_ref,
    bufs,
    buft,
    sems,
    lab_scr,
    acc_s,
    acc_t,
    cnt,
    *,
    nsteps,
    h_per_b,
    ratio,
):
    i = pl.program_id(0)
    nbuf, rows = bufs.shape[0], bufs.shape[1]
    steps_per_b = h_per_b // rows
    h0 = (i % steps_per_b) * rows
    b = i // steps_per_b
    slot = i % nbuf
    f32 = jnp.float32
    bf16 = jnp.bfloat16

    def _start_group(step, slot_idx):
        gh = (step % steps_per_b) * rows
        gb = step // steps_per_b
        for r in range(rows):
            pltpu.make_async_copy(
                fs_ref.at[gb, :, gh + r, :], bufs.at[slot_idx, r], sems.at[slot_idx, r, 0]
            ).start()
            pltpu.make_async_copy(
                ft_ref.at[gb, :, gh + r, :], buft.at[slot_idx, r], sems.at[slot_idx, r, 1]
            ).start()

    @pl.when(i == 0)
    def _init():
        acc_s[...] = jnp.zeros_like(acc_s)
        acc_t[...] = jnp.zeros_like(acc_t)
        cnt[...] = jnp.zeros_like(cnt)
        for j in range(nbuf - 1):
            _start_group(j, j)

    nxt = i + nbuf - 1

    @pl.when(nxt < nsteps)
    def _prefetch():
        _start_group(nxt, nxt % nbuf)

    @pl.when(i % steps_per_b == 0)
    def _prep_labels():
        lab = lab_ref[...].astype(bf16)
        lh, lw = lab.shape
        w = lw // ratio
        hh = lh // ratio
        jc = jax.lax.broadcasted_iota(jnp.int32, (lw, w), 0)
        wc = jax.lax.broadcasted_iota(jnp.int32, (lw, w), 1)
        sel_col = (jc == ratio * wc).astype(bf16)
        b2 = jax.lax.dot(lab, sel_col, preferred_element_type=f32).astype(
            bf16
        )
        rr = jax.lax.broadcasted_iota(jnp.int32, (hh, lh), 0)
        jr = jax.lax.broadcasted_iota(jnp.int32, (hh, lh), 1)
        sel_row = (jr == ratio * rr).astype(bf16)
        lab_scr[...] = jax.lax.dot(sel_row, b2, preferred_element_type=f32)

    w = bufs.shape[3]
    kio = jax.lax.broadcasted_iota(jnp.int32, (_PAD, w), 0).astype(f32)
    dims = (((1,), (1,)), ((), ()))

    ps = None
    pt = None
    csum = None
    for r in range(rows):
        pltpu.make_async_copy(
            fs_ref.at[b, :, h0 + r, :], bufs.at[slot, r], sems.at[slot, r, 0]
        ).wait()
        pltpu.make_async_copy(
            ft_ref.at[b, :, h0 + r, :], buft.at[slot, r], sems.at[slot, r, 1]
        ).wait()
        x_s = bufs[slot, r]
        x_t = buft[slot, r]

        row = lab_scr[pl.ds(h0 + r, 1), :]
        oh_t_f = (jnp.broadcast_to(row, (_PAD, w)) == kio).astype(f32)
        oh_t = oh_t_f.astype(bf16)

        s_hi = x_s.astype(bf16)
        s_lo = (x_s - s_hi.astype(f32)).astype(bf16)
        t_hi = x_t.astype(bf16)
        t_lo = (x_t - t_hi.astype(f32)).astype(bf16)

        ds = jax.lax.dot_general(
            s_hi, oh_t, dims, preferred_element_type=f32
        ) + jax.lax.dot_general(s_lo, oh_t, dims, preferred_element_type=f32)
        dt = jax.lax.dot_general(
            t_hi, oh_t, dims, preferred_element_type=f32
        ) + jax.lax.dot_general(t_lo, oh_t, dims, preferred_element_type=f32)
        ps = ds if ps is None else ps + ds
        pt = dt if pt is None else pt + dt
        csum = oh_t_f if csum is None else csum + oh_t_f
    acc_s[...] += ps
    acc_t[...] += pt
    cnt[...] += csum

    @pl.when(i == nsteps - 1)
    def _fin():
        hp = jax.lax.Precision.HIGHEST
        counts = jnp.sum(cnt[...], axis=1)
        present = counts > 0.0
        denom = jnp.where(present, counts, 1.0)
        mean_s = acc_s[...] / denom[None, :]
        mean_t = acc_t[...] / denom[None, :]
        ns = jnp.sqrt(jnp.sum(mean_s * mean_s, axis=0, keepdims=True))
        nt = jnp.sqrt(jnp.sum(mean_t * mean_t, axis=0, keepdims=True))
        s_n = jnp.where(present[None, :], mean_s / jnp.maximum(ns, 1e-12), 0.0)
        t_n = jnp.where(present[None, :], mean_t / jnp.maximum(nt, 1e-12), 0.0)
        logits = (
            jax.lax.dot_general(s_n, t_n, (((0,), (0,)), ((), ())), precision=hp)
            / _TEMP
        )
        ri = jax.lax.broadcasted_iota(jnp.int32, (_PAD, _PAD), 0)
        ci = jax.lax.broadcasted_iota(jnp.int32, (_PAD, _PAD), 1)
        e = jnp.where(ci < _NCLS, jnp.exp(logits), 0.0)
        row_sum = jnp.sum(e, axis=1)
        diag = jnp.sum(jnp.where(ri == ci, logits, 0.0), axis=1)
        per_cls = jnp.log(row_sum) - diag
        n_present = jnp.maximum(jnp.sum(jnp.where(present, 1.0, 0.0)), 1.0)
        loss = jnp.sum(jnp.where(present, per_cls, 0.0)) / n_present
        out_ref[...] = jnp.broadcast_to(loss, (1, 1))


@jax.jit
def kernel(features_s, features_t, labels):
    B, C, H, W = features_s.shape
    Lh, Lw = labels.shape[1], labels.shape[2]
    ratio = Lh // H
    rows = 32
    nbuf = 4
    steps_per_b = H // rows
    nsteps = B * steps_per_b

    out = pl.pallas_call(
        functools.partial(_body, nsteps=nsteps, h_per_b=H, ratio=ratio),
        grid=(nsteps,),
        in_specs=[
            pl.BlockSpec(memory_space=pltpu.MemorySpace.HBM),
            pl.BlockSpec(memory_space=pltpu.MemorySpace.HBM),
            pl.BlockSpec((None, Lh, Lw), lambda i: (i // steps_per_b, 0, 0)),
        ],
        out_specs=pl.BlockSpec((1, 1), lambda i: (0, 0)),
        out_shape=jax.ShapeDtypeStruct((1, 1), jnp.float32),
        scratch_shapes=[
            pltpu.VMEM((nbuf, rows, C, W), jnp.float32),
            pltpu.VMEM((nbuf, rows, C, W), jnp.float32),
            pltpu.SemaphoreType.DMA((nbuf, rows, 2)),
            pltpu.VMEM((H, W), jnp.float32),
            pltpu.VMEM((C, _PAD), jnp.float32),
            pltpu.VMEM((C, _PAD), jnp.float32),
            pltpu.VMEM((_PAD, W), jnp.float32),
        ],
    )(features_s, features_t, labels)
    return out[0, 0]


# speedup vs baseline: 6.0073x; 1.0680x over previous
import functools

import jax
import jax.numpy as jnp
from jax.experimental import pallas as pl
from jax.experimental.pallas import tpu as pltpu

_NCLS = 19
_PAD = 32
_TEMP = 0.1


def _body(
    fs_ref,
    ft_ref,
    lab_ref,
    out_ref,
    bufs,
    buft,
    sems,
    lab_scr,
    acc_s,
    acc_t,
    cnt,
    *,
    nsteps,
    h_per_b,
    ratio,
):
    i = pl.program_id(0)
    nbuf, rows = bufs.shape[0], bufs.shape[1]
    steps_per_b = h_per_b // rows
    h0 = (i % steps_per_b) * rows
    b = i // steps_per_b
    slot = i % nbuf
    f32 = jnp.float32
    bf16 = jnp.bfloat16

    def _start_group(step, slot_idx):
        gh = (step % steps_per_b) * rows
        gb = step // steps_per_b
        for r in range(rows):
            pltpu.make_async_copy(
                fs_ref.at[gb, :, gh + r, :], bufs.at[slot_idx, r], sems.at[slot_idx, r, 0]
            ).start()
            pltpu.make_async_copy(
                ft_ref.at[gb, :, gh + r, :], buft.at[slot_idx, r], sems.at[slot_idx, r, 1]
            ).start()

    @pl.when(i == 0)
    def _init():
        acc_s[...] = jnp.zeros_like(acc_s)
        acc_t[...] = jnp.zeros_like(acc_t)
        cnt[...] = jnp.zeros_like(cnt)
        for j in range(nbuf - 1):
            _start_group(j, j)

    nxt = i + nbuf - 1

    @pl.when(nxt < nsteps)
    def _prefetch():
        _start_group(nxt, nxt % nbuf)

    @pl.when(i % steps_per_b == 0)
    def _prep_labels():
        lab = lab_ref[...].astype(bf16)
        lh, lw = lab.shape
        w = lw // ratio
        hh = lh // ratio
        jc = jax.lax.broadcasted_iota(jnp.int32, (lw, w), 0)
        wc = jax.lax.broadcasted_iota(jnp.int32, (lw, w), 1)
        sel_col = (jc == ratio * wc).astype(bf16)
        b2 = jax.lax.dot(lab, sel_col, preferred_element_type=f32).astype(
            bf16
        )
        rr = jax.lax.broadcasted_iota(jnp.int32, (hh, lh), 0)
        jr = jax.lax.broadcasted_iota(jnp.int32, (hh, lh), 1)
        sel_row = (jr == ratio * rr).astype(bf16)
        lab_scr[...] = jax.lax.dot(sel_row, b2, preferred_element_type=f32)

    w = bufs.shape[3]
    kio = jax.lax.broadcasted_iota(jnp.int32, (_PAD, w), 0).astype(f32)
    dims = (((1,), (1,)), ((), ()))

    ps = None
    pt = None
    csum = None
    for r in range(rows):
        pltpu.make_async_copy(
            fs_ref.at[b, :, h0 + r, :], bufs.at[slot, r], sems.at[slot, r, 0]
        ).wait()
        pltpu.make_async_copy(
            ft_ref.at[b, :, h0 + r, :], buft.at[slot, r], sems.at[slot, r, 1]
        ).wait()
        x_s = bufs[slot, r]
        x_t = buft[slot, r]

        row = lab_scr[pl.ds(h0 + r, 1), :]
        oh_t_f = (jnp.broadcast_to(row, (_PAD, w)) == kio).astype(f32)
        oh_t = oh_t_f.astype(bf16)

        s_hi = x_s.astype(bf16)
        s_lo = (x_s - s_hi.astype(f32)).astype(bf16)
        t_hi = x_t.astype(bf16)
        t_lo = (x_t - t_hi.astype(f32)).astype(bf16)

        ds = jax.lax.dot_general(
            s_hi, oh_t, dims, preferred_element_type=f32
        ) + jax.lax.dot_general(s_lo, oh_t, dims, preferred_element_type=f32)
        dt = jax.lax.dot_general(
            t_hi, oh_t, dims, preferred_element_type=f32
        ) + jax.lax.dot_general(t_lo, oh_t, dims, preferred_element_type=f32)
        ps = ds if ps is None else ps + ds
        pt = dt if pt is None else pt + dt
        csum = oh_t_f if csum is None else csum + oh_t_f
    acc_s[...] += ps
    acc_t[...] += pt
    cnt[...] += csum

    @pl.when(i == nsteps - 1)
    def _fin():
        hp = jax.lax.Precision.HIGHEST
        counts = jnp.sum(cnt[...], axis=1)
        present = counts > 0.0
        denom = jnp.where(present, counts, 1.0)
        mean_s = acc_s[...] / denom[None, :]
        mean_t = acc_t[...] / denom[None, :]
        ns = jnp.sqrt(jnp.sum(mean_s * mean_s, axis=0, keepdims=True))
        nt = jnp.sqrt(jnp.sum(mean_t * mean_t, axis=0, keepdims=True))
        s_n = jnp.where(present[None, :], mean_s / jnp.maximum(ns, 1e-12), 0.0)
        t_n = jnp.where(present[None, :], mean_t / jnp.maximum(nt, 1e-12), 0.0)
        logits = (
            jax.lax.dot_general(s_n, t_n, (((0,), (0,)), ((), ())), precision=hp)
            / _TEMP
        )
        ri = jax.lax.broadcasted_iota(jnp.int32, (_PAD, _PAD), 0)
        ci = jax.lax.broadcasted_iota(jnp.int32, (_PAD, _PAD), 1)
        e = jnp.where(ci < _NCLS, jnp.exp(logits), 0.0)
        row_sum = jnp.sum(e, axis=1)
        diag = jnp.sum(jnp.where(ri == ci, logits, 0.0), axis=1)
        per_cls = jnp.log(row_sum) - diag
        n_present = jnp.maximum(jnp.sum(jnp.where(present, 1.0, 0.0)), 1.0)
        loss = jnp.sum(jnp.where(present, per_cls, 0.0)) / n_present
        out_ref[...] = jnp.broadcast_to(loss, (1, 1))


@jax.jit
def kernel(features_s, features_t, labels):
    B, C, H, W = features_s.shape
    Lh, Lw = labels.shape[1], labels.shape[2]
    ratio = Lh // H
    rows = 64
    nbuf = 2
    steps_per_b = H // rows
    nsteps = B * steps_per_b

    out = pl.pallas_call(
        functools.partial(_body, nsteps=nsteps, h_per_b=H, ratio=ratio),
        grid=(nsteps,),
        in_specs=[
            pl.BlockSpec(memory_space=pltpu.MemorySpace.HBM),
            pl.BlockSpec(memory_space=pltpu.MemorySpace.HBM),
            pl.BlockSpec((None, Lh, Lw), lambda i: (i // steps_per_b, 0, 0)),
        ],
        out_specs=pl.BlockSpec((1, 1), lambda i: (0, 0)),
        out_shape=jax.ShapeDtypeStruct((1, 1), jnp.float32),
        scratch_shapes=[
            pltpu.VMEM((nbuf, rows, C, W), jnp.float32),
            pltpu.VMEM((nbuf, rows, C, W), jnp.float32),
            pltpu.SemaphoreType.DMA((nbuf, rows, 2)),
            pltpu.VMEM((H, W), jnp.float32),
            pltpu.VMEM((C, _PAD), jnp.float32),
            pltpu.VMEM((C, _PAD), jnp.float32),
            pltpu.VMEM((_PAD, W), jnp.float32),
        ],
    )(features_s, features_t, labels)
    return out[0, 0]
